# baseline (speedup 1.0000x reference)
import functools

import jax
import jax.numpy as jnp
from jax import lax
from jax.experimental import pallas as pl
from jax.experimental.pallas import tpu as pltpu
from jax.experimental.pallas import tpu_sc as plsc

NUM_NEIGHBOURS = 50
DELTA = 1e-3

A = 8
C = 100000
CPAD = 102400
BC = 12800
NBLK = CPAD // BC
Q = 64
D = 64
SPB = BC // 128
NSLAB = CPAD // 128
GPAD = 128
NG = NBLK * GPAD
SEL = 64
K = NUM_NEIGHBOURS
PAD_VAL = 3.0e7
BIG = 3.0e38

NW = 32
ROWS = A * Q
RPW = ROWS // NW


def _dist_kernel(obs_ref, w_ref, b_ref, keys_ref, d2_ref, gm_ref):
    j = pl.program_id(1)
    h = jnp.dot(obs_ref[...], w_ref[...],
                preferred_element_type=jnp.float32) + b_ref[...][None, :]
    kt = keys_ref[0]
    k2 = jnp.sum(kt * kt, axis=0)
    q2 = jnp.sum(h * h, axis=1, keepdims=True)
    hk = lax.dot_general(h, kt, (((1,), (0,)), ((), ())),
                         preferred_element_type=jnp.float32)
    d2 = q2 - 2.0 * hk + k2[None, :]
    ci = lax.broadcasted_iota(jnp.int32, (Q, BC), 1) + j * BC
    d2 = jnp.where(ci < C, d2, BIG)
    d2_ref[0] = d2
    m1 = jnp.min(d2.reshape(Q, SPB, 128), axis=-1)
    gm_ref[0, 0] = jnp.concatenate(
        [m1, jnp.full((Q, GPAD - SPB), BIG, jnp.float32)], axis=1)


def _distances(observations, W_emb, b_emb, dnd_keys):
    keys_t = jnp.swapaxes(dnd_keys, 1, 2)
    return pl.pallas_call(
        _dist_kernel,
        grid=(A, NBLK),
        in_specs=[
            pl.BlockSpec((Q, 128), lambda a, j: (0, 0)),
            pl.BlockSpec((128, D), lambda a, j: (0, 0)),
            pl.BlockSpec((D,), lambda a, j: (0,)),
            pl.BlockSpec((1, D, BC), lambda a, j: (a, 0, j)),
        ],
        out_specs=[
            pl.BlockSpec((1, Q, BC), lambda a, j: (a, 0, j)),
            pl.BlockSpec((1, 1, Q, GPAD), lambda a, j: (a, j, 0, 0)),
        ],
        out_shape=[
            jax.ShapeDtypeStruct((A, Q, CPAD), jnp.float32),
            jax.ShapeDtypeStruct((A, NBLK, Q, GPAD), jnp.float32),
        ],
    )(observations, W_emb, b_emb, keys_t)


def _vt_kernel(vt_ref, out_ref):
    vt = vt_ref[0]
    eye = (lax.broadcasted_iota(jnp.int32, (D, D), 0)
           == lax.broadcasted_iota(jnp.int32, (D, D), 1)).astype(jnp.float32)
    out_ref[0] = lax.dot_general(vt, eye, (((0,), (0,)), ((), ())),
                                 preferred_element_type=jnp.float32)


def _transpose_values(dnd_values):
    vals_t = jnp.swapaxes(dnd_values, 1, 2)
    return pl.pallas_call(
        _vt_kernel,
        grid=(A, NBLK),
        in_specs=[pl.BlockSpec((1, D, BC), lambda a, j: (a, 0, j))],
        out_specs=pl.BlockSpec((1, BC, D), lambda a, j: (a, j, 0)),
        out_shape=jax.ShapeDtypeStruct((A, CPAD, D), jnp.float32),
    )(vals_t)


def _scal(x):
    return x if getattr(x, "ndim", 0) == 0 else jnp.max(x)


def _ffs(b):
    return _scal(plsc.all_reduce_ffs(b))


def _sc_body(d2_hbm, gm_hbm, vals_hbm, out_hbm,
             cand_v, gm_v, cm1_v, cm_v, vidx_v, vbuf, obuf, wbuf_v,
             selc_smem, vcol_smem, sem, sem2):
    lane = lax.iota(jnp.int32, 16)
    lane0 = lane == 0

    def set1f(ref, pos, val):
        plsc.store_scatter(ref, [jnp.full((16,), pos, jnp.int32)],
                           jnp.full((16,), val, jnp.float32), mask=lane0)

    def set1i(ref, pos, val):
        plsc.store_scatter(ref, [jnp.full((16,), pos, jnp.int32)],
                           jnp.full((16,), val, jnp.int32), mask=lane0)

    wid = lax.axis_index("s") * 2 + lax.axis_index("c")

    def row_body(r, _):
        row = wid * RPW + r
        a = row >> 6
        q = row & (Q - 1)

        for j in range(NBLK):
            pltpu.sync_copy(gm_hbm.at[a, j, q],
                            gm_v.at[pl.ds(j * GPAD, GPAD)])

        for v in range(NG // 16):
            set1f(cm1_v, v, jnp.min(gm_v[pl.ds(v * 16, 16)]))

        def ext1(i, _c):
            v0 = cm1_v[pl.ds(0, 16)]
            v1 = cm1_v[pl.ds(16, 16)]
            v2 = cm1_v[pl.ds(32, 16)]
            v3 = cm1_v[pl.ds(48, 16)]
            m = jnp.min(jnp.minimum(jnp.minimum(v0, v1),
                                    jnp.minimum(v2, v3)))
            f0 = _ffs(v0 == m)
            f1 = _ffs(v1 == m)
            f2 = _ffs(v2 == m)
            f3 = _ffs(v3 == m)
            s3 = jnp.where(f0 < 16, f0,
                           jnp.where(f1 < 16, 16 + f1,
                                     jnp.where(f2 < 16, 32 + f2, 48 + f3)))
            gv = gm_v[pl.ds(s3 * 16, 16)]
            l1 = _ffs(gv == m)
            gidx = s3 * 16 + l1
            jb = gidx >> 7
            u = gidx & (GPAD - 1)
            cb = jb * BC + u * 128
            selc_smem[i] = cb
            pltpu.async_copy(d2_hbm.at[a, q, pl.ds(cb, 128)],
                             cand_v.at[i], sem)
            set1f(cm_v, i, m)
            set1f(gm_v, gidx, BIG)
            set1f(cm1_v, s3, jnp.min(gm_v[pl.ds(s3 * 16, 16)]))
            return 0

        lax.fori_loop(0, SEL, ext1, 0)

        for v in range(SEL // 16):
            vidx_v[pl.ds(v * 16, 16)] = (a * CPAD + lane + v * 16) >> 1
            wbuf_v[pl.ds(v * 16, 16)] = jnp.zeros((16,), jnp.float32)
            for t in range(16):
                vcol_smem[v * 16 + t] = 0

        pltpu.make_async_copy(d2_hbm.at[0, pl.ds(0, SEL), pl.ds(0, 128)],
                              cand_v, sem).wait()

        def ext2(i, _c):
            c0 = cm_v[pl.ds(0, 16)]
            c1 = cm_v[pl.ds(16, 16)]
            c2 = cm_v[pl.ds(32, 16)]
            c3 = cm_v[pl.ds(48, 16)]
            m = jnp.min(jnp.minimum(jnp.minimum(c0, c1),
                                    jnp.minimum(c2, c3)))
            f0 = _ffs(c0 == m)
            f1 = _ffs(c1 == m)
            f2 = _ffs(c2 == m)
            f3 = _ffs(c3 == m)
            s = jnp.where(f0 < 16, f0,
                          jnp.where(f1 < 16, 16 + f1,
                                    jnp.where(f2 < 16, 32 + f2, 48 + f3)))
            cb = selc_smem[s]
            srow = jnp.full((16,), s, jnp.int32)
            col = jnp.int32(-1)
            for p in range(8):
                cv = plsc.load_gather(cand_v, [srow, lane + p * 16])
                fp = _ffs(cv == m)
                col = jnp.where((col < 0) & (fp < 16), p * 16 + fp, col)
            c_local = cb + col
            v_glob = a * CPAD + jnp.minimum(c_local, C - 1)
            set1i(vidx_v, i, v_glob >> 1)
            vcol_smem[i] = (v_glob & 1) * 64
            wv = 1.0 / (jnp.full((16,), jnp.maximum(m, 0.0) + DELTA,
                                 jnp.float32))
            plsc.store_scatter(wbuf_v, [jnp.full((16,), i, jnp.int32)],
                               wv, mask=lane0)
            plsc.store_scatter(cand_v, [srow, jnp.full((16,), col, jnp.int32)],
                               jnp.full((16,), BIG, jnp.float32), mask=lane0)
            nm = plsc.load_gather(cand_v, [srow, lane])
            for p in range(1, 8):
                nm = jnp.minimum(nm,
                                 plsc.load_gather(cand_v, [srow, lane + p * 16]))
            set1f(cm_v, s, jnp.min(nm))
            return 0

        lax.fori_loop(0, K, ext2, 0)

        pltpu.async_copy(vals_hbm.at[vidx_v], vbuf, sem).wait()

        def wsloop(i, acc):
            irow = jnp.full((16,), i, jnp.int32)
            wv = plsc.load_gather(wbuf_v, [irow])
            colb = vcol_smem[i]
            return tuple(
                acc[dv] + wv * plsc.load_gather(vbuf,
                                                [irow, colb + lane + dv * 16])
                for dv in range(4))

        z = jnp.zeros((16,), jnp.float32)
        acc = lax.fori_loop(0, SEL, wsloop, (z, z, z, z))
        ws4 = (wbuf_v[pl.ds(0, 16)] + wbuf_v[pl.ds(16, 16)]
               + wbuf_v[pl.ds(32, 16)] + wbuf_v[pl.ds(48, 16)])
        invv = 1.0 / jnp.full((16,), jnp.sum(ws4), jnp.float32)
        for dv in range(4):
            obuf[pl.ds(dv * 16, 16)] = acc[dv] * invv
        pltpu.sync_copy(obuf, out_hbm.at[pl.ds((q * A + a) * D, D)])
        return 0

    lax.fori_loop(0, RPW, row_body, 0)


def _sc_topk(d2, gm, vals):
    f = functools.partial(
        pl.kernel,
        mesh=plsc.VectorSubcoreMesh(core_axis_name="c", subcore_axis_name="s"),
        compiler_params=pltpu.CompilerParams(needs_layout_passes=False,
                                             use_tc_tiling_on_sc=True),
        out_type=jax.ShapeDtypeStruct((Q * A * D,), jnp.float32),
        scratch_types=[
            pltpu.VMEM((SEL, 128), jnp.float32),
            pltpu.VMEM((NG,), jnp.float32),
            pltpu.VMEM((NG // 16,), jnp.float32),
            pltpu.VMEM((SEL,), jnp.float32),
            pltpu.VMEM((SEL,), jnp.int32),
            pltpu.VMEM((SEL, 128), jnp.float32),
            pltpu.VMEM((D,), jnp.float32),
            pltpu.VMEM((SEL,), jnp.float32),
            pltpu.SMEM((SEL,), jnp.int32),
            pltpu.SMEM((SEL,), jnp.int32),
            pltpu.SemaphoreType.DMA,
            pltpu.SemaphoreType.DMA,
        ],
    )(_sc_body)
    return f(d2, gm, vals)


def kernel(observations, W_emb, b_emb, dnd_keys, dnd_values):
    d2, gm = _distances(observations, W_emb, b_emb, dnd_keys)
    vals_tr = _transpose_values(dnd_values)
    out = _sc_topk(d2, gm, vals_tr.reshape(A * CPAD * D // 128, 128))
    return out.reshape(Q, A, D)


# speedup vs baseline: 1.0023x; 1.0023x over previous
import functools

import jax
import jax.numpy as jnp
from jax import lax
from jax.experimental import pallas as pl
from jax.experimental.pallas import tpu as pltpu
from jax.experimental.pallas import tpu_sc as plsc

NUM_NEIGHBOURS = 50
DELTA = 1e-3

A = 8
C = 100000
CPAD = 102400
BC = 12800
NBLK = CPAD // BC
Q = 64
D = 64
SPB = BC // 128
NSLAB = CPAD // 128
GPAD = 128
NG = NBLK * GPAD
SEL = 64
K = NUM_NEIGHBOURS
PAD_VAL = 3.0e7
BIG = 3.0e38

NW = 32
ROWS = A * Q
RPW = ROWS // NW


def _dist_kernel(obs_ref, w_ref, b_ref, keys_ref, d2_ref, gm_ref):
    j = pl.program_id(1)
    h = jnp.dot(obs_ref[...], w_ref[...],
                preferred_element_type=jnp.float32) + b_ref[...][None, :]
    kt = keys_ref[0]
    k2 = jnp.sum(kt * kt, axis=0)
    q2 = jnp.sum(h * h, axis=1, keepdims=True)
    hk = lax.dot_general(h, kt, (((1,), (0,)), ((), ())),
                         preferred_element_type=jnp.float32)
    d2 = q2 - 2.0 * hk + k2[None, :]
    ci = lax.broadcasted_iota(jnp.int32, (Q, BC), 1) + j * BC
    d2 = jnp.where(ci < C, d2, BIG)
    d2_ref[0] = d2
    m1 = jnp.min(d2.reshape(Q, SPB, 128), axis=-1)
    gm_ref[0, 0] = jnp.concatenate(
        [m1, jnp.full((Q, GPAD - SPB), BIG, jnp.float32)], axis=1)


def _distances(observations, W_emb, b_emb, dnd_keys):
    keys_t = jnp.swapaxes(dnd_keys, 1, 2)
    return pl.pallas_call(
        _dist_kernel,
        grid=(A, NBLK),
        in_specs=[
            pl.BlockSpec((Q, 128), lambda a, j: (0, 0)),
            pl.BlockSpec((128, D), lambda a, j: (0, 0)),
            pl.BlockSpec((D,), lambda a, j: (0,)),
            pl.BlockSpec((1, D, BC), lambda a, j: (a, 0, j)),
        ],
        out_specs=[
            pl.BlockSpec((1, Q, BC), lambda a, j: (a, 0, j)),
            pl.BlockSpec((1, 1, Q, GPAD), lambda a, j: (a, j, 0, 0)),
        ],
        out_shape=[
            jax.ShapeDtypeStruct((A, Q, CPAD), jnp.float32),
            jax.ShapeDtypeStruct((A, NBLK, Q, GPAD), jnp.float32),
        ],
    )(observations, W_emb, b_emb, keys_t)


def _vt_kernel(vt_ref, out_ref):
    out_ref[0] = jnp.swapaxes(vt_ref[0], 0, 1)


def _transpose_values(dnd_values):
    vals_t = jnp.swapaxes(dnd_values, 1, 2)
    return pl.pallas_call(
        _vt_kernel,
        grid=(A, NBLK),
        in_specs=[pl.BlockSpec((1, D, BC), lambda a, j: (a, 0, j))],
        out_specs=pl.BlockSpec((1, BC, D), lambda a, j: (a, j, 0)),
        out_shape=jax.ShapeDtypeStruct((A, CPAD, D), jnp.float32),
    )(vals_t)


def _scal(x):
    return x if getattr(x, "ndim", 0) == 0 else jnp.max(x)


def _ffs(b):
    return _scal(plsc.all_reduce_ffs(b))


def _sc_body(d2_hbm, gm_hbm, vals_hbm, out_hbm,
             cand_v, gm_v, cm1_v, cm_v, vidx_v, vbuf, obuf, wbuf_v,
             selc_smem, vcol_smem, sem, sem2):
    lane = lax.iota(jnp.int32, 16)
    lane0 = lane == 0

    def set1f(ref, pos, val):
        plsc.store_scatter(ref, [jnp.full((16,), pos, jnp.int32)],
                           jnp.full((16,), val, jnp.float32), mask=lane0)

    def set1i(ref, pos, val):
        plsc.store_scatter(ref, [jnp.full((16,), pos, jnp.int32)],
                           jnp.full((16,), val, jnp.int32), mask=lane0)

    wid = lax.axis_index("s") * 2 + lax.axis_index("c")

    def row_body(r, _):
        row = wid * RPW + r
        a = row >> 6
        q = row & (Q - 1)

        for j in range(NBLK):
            pltpu.sync_copy(gm_hbm.at[a, j, q],
                            gm_v.at[pl.ds(j * GPAD, GPAD)])

        for v in range(NG // 16):
            set1f(cm1_v, v, jnp.min(gm_v[pl.ds(v * 16, 16)]))

        def ext1(i, _c):
            v0 = cm1_v[pl.ds(0, 16)]
            v1 = cm1_v[pl.ds(16, 16)]
            v2 = cm1_v[pl.ds(32, 16)]
            v3 = cm1_v[pl.ds(48, 16)]
            m = jnp.min(jnp.minimum(jnp.minimum(v0, v1),
                                    jnp.minimum(v2, v3)))
            f0 = _ffs(v0 == m)
            f1 = _ffs(v1 == m)
            f2 = _ffs(v2 == m)
            f3 = _ffs(v3 == m)
            s3 = jnp.where(f0 < 16, f0,
                           jnp.where(f1 < 16, 16 + f1,
                                     jnp.where(f2 < 16, 32 + f2, 48 + f3)))
            gv = gm_v[pl.ds(s3 * 16, 16)]
            l1 = _ffs(gv == m)
            gidx = s3 * 16 + l1
            jb = gidx >> 7
            u = gidx & (GPAD - 1)
            cb = jb * BC + u * 128
            selc_smem[i] = cb
            pltpu.async_copy(d2_hbm.at[a, q, pl.ds(cb, 128)],
                             cand_v.at[i], sem)
            set1f(cm_v, i, m)
            set1f(gm_v, gidx, BIG)
            set1f(cm1_v, s3, jnp.min(gm_v[pl.ds(s3 * 16, 16)]))
            return 0

        lax.fori_loop(0, SEL, ext1, 0)

        for v in range(SEL // 16):
            vidx_v[pl.ds(v * 16, 16)] = (a * CPAD + lane + v * 16) >> 1
            wbuf_v[pl.ds(v * 16, 16)] = jnp.zeros((16,), jnp.float32)
            for t in range(16):
                vcol_smem[v * 16 + t] = 0

        pltpu.make_async_copy(d2_hbm.at[0, pl.ds(0, SEL), pl.ds(0, 128)],
                              cand_v, sem).wait()

        def ext2(i, _c):
            c0 = cm_v[pl.ds(0, 16)]
            c1 = cm_v[pl.ds(16, 16)]
            c2 = cm_v[pl.ds(32, 16)]
            c3 = cm_v[pl.ds(48, 16)]
            m = jnp.min(jnp.minimum(jnp.minimum(c0, c1),
                                    jnp.minimum(c2, c3)))
            f0 = _ffs(c0 == m)
            f1 = _ffs(c1 == m)
            f2 = _ffs(c2 == m)
            f3 = _ffs(c3 == m)
            s = jnp.where(f0 < 16, f0,
                          jnp.where(f1 < 16, 16 + f1,
                                    jnp.where(f2 < 16, 32 + f2, 48 + f3)))
            cb = selc_smem[s]
            srow = jnp.full((16,), s, jnp.int32)
            col = jnp.int32(-1)
            for p in range(8):
                cv = plsc.load_gather(cand_v, [srow, lane + p * 16])
                fp = _ffs(cv == m)
                col = jnp.where((col < 0) & (fp < 16), p * 16 + fp, col)
            c_local = cb + col
            v_glob = a * CPAD + jnp.minimum(c_local, C - 1)
            set1i(vidx_v, i, v_glob >> 1)
            vcol_smem[i] = (v_glob & 1) * 64
            wv = 1.0 / (jnp.full((16,), jnp.maximum(m, 0.0) + DELTA,
                                 jnp.float32))
            plsc.store_scatter(wbuf_v, [jnp.full((16,), i, jnp.int32)],
                               wv, mask=lane0)
            plsc.store_scatter(cand_v, [srow, jnp.full((16,), col, jnp.int32)],
                               jnp.full((16,), BIG, jnp.float32), mask=lane0)
            nm = plsc.load_gather(cand_v, [srow, lane])
            for p in range(1, 8):
                nm = jnp.minimum(nm,
                                 plsc.load_gather(cand_v, [srow, lane + p * 16]))
            set1f(cm_v, s, jnp.min(nm))
            return 0

        lax.fori_loop(0, K, ext2, 0)

        pltpu.async_copy(vals_hbm.at[vidx_v], vbuf, sem).wait()

        def wsloop(i, acc):
            irow = jnp.full((16,), i, jnp.int32)
            wv = plsc.load_gather(wbuf_v, [irow])
            colb = vcol_smem[i]
            return tuple(
                acc[dv] + wv * plsc.load_gather(vbuf,
                                                [irow, colb + lane + dv * 16])
                for dv in range(4))

        z = jnp.zeros((16,), jnp.float32)
        acc = lax.fori_loop(0, SEL, wsloop, (z, z, z, z))
        ws4 = (wbuf_v[pl.ds(0, 16)] + wbuf_v[pl.ds(16, 16)]
               + wbuf_v[pl.ds(32, 16)] + wbuf_v[pl.ds(48, 16)])
        invv = 1.0 / jnp.full((16,), jnp.sum(ws4), jnp.float32)
        for dv in range(4):
            obuf[pl.ds(dv * 16, 16)] = acc[dv] * invv
        pltpu.sync_copy(obuf, out_hbm.at[pl.ds((q * A + a) * D, D)])
        return 0

    lax.fori_loop(0, RPW, row_body, 0)


def _sc_topk(d2, gm, vals):
    f = functools.partial(
        pl.kernel,
        mesh=plsc.VectorSubcoreMesh(core_axis_name="c", subcore_axis_name="s"),
        compiler_params=pltpu.CompilerParams(needs_layout_passes=False,
                                             use_tc_tiling_on_sc=True),
        out_type=jax.ShapeDtypeStruct((Q * A * D,), jnp.float32),
        scratch_types=[
            pltpu.VMEM((SEL, 128), jnp.float32),
            pltpu.VMEM((NG,), jnp.float32),
            pltpu.VMEM((NG // 16,), jnp.float32),
            pltpu.VMEM((SEL,), jnp.float32),
            pltpu.VMEM((SEL,), jnp.int32),
            pltpu.VMEM((SEL, 128), jnp.float32),
            pltpu.VMEM((D,), jnp.float32),
            pltpu.VMEM((SEL,), jnp.float32),
            pltpu.SMEM((SEL,), jnp.int32),
            pltpu.SMEM((SEL,), jnp.int32),
            pltpu.SemaphoreType.DMA,
            pltpu.SemaphoreType.DMA,
        ],
    )(_sc_body)
    return f(d2, gm, vals)


def kernel(observations, W_emb, b_emb, dnd_keys, dnd_values):
    d2, gm = _distances(observations, W_emb, b_emb, dnd_keys)
    vals_tr = _transpose_values(dnd_values)
    out = _sc_topk(d2, gm, vals_tr.reshape(A * CPAD * D // 128, 128))
    return out.reshape(Q, A, D)


# speedup vs baseline: 1.5183x; 1.5149x over previous
import functools

import jax
import jax.numpy as jnp
from jax import lax
from jax.experimental import pallas as pl
from jax.experimental.pallas import tpu as pltpu
from jax.experimental.pallas import tpu_sc as plsc

NUM_NEIGHBOURS = 50
DELTA = 1e-3

A = 8
C = 100000
CPAD = 102400
BC = 12800
NBLK = CPAD // BC
Q = 64
D = 64
SPB = BC // 128
NSLAB = CPAD // 128
GPAD = 128
NG = NBLK * GPAD
SEL = 64
K = NUM_NEIGHBOURS
PAD_VAL = 3.0e7
BIG = 3.0e38

NW = 32
ROWS = A * Q
RPW = ROWS // NW


def _dist_kernel(obs_ref, w_ref, b_ref, keys_ref, d2_ref, gm_ref):
    j = pl.program_id(1)
    h = jnp.dot(obs_ref[...], w_ref[...],
                preferred_element_type=jnp.float32) + b_ref[...][None, :]
    kt = keys_ref[0]
    k2 = jnp.sum(kt * kt, axis=0)
    q2 = jnp.sum(h * h, axis=1, keepdims=True)
    hk = lax.dot_general(h, kt, (((1,), (0,)), ((), ())),
                         preferred_element_type=jnp.float32)
    d2 = q2 - 2.0 * hk + k2[None, :]
    ci = lax.broadcasted_iota(jnp.int32, (Q, BC), 1) + j * BC
    d2 = jnp.where(ci < C, d2, BIG)
    d2_ref[0] = d2
    m1 = jnp.min(d2.reshape(Q, SPB, 128), axis=-1)
    gm_ref[0, 0] = jnp.concatenate(
        [m1, jnp.full((Q, GPAD - SPB), BIG, jnp.float32)], axis=1)


def _distances(observations, W_emb, b_emb, dnd_keys):
    keys_t = jnp.swapaxes(dnd_keys, 1, 2)
    return pl.pallas_call(
        _dist_kernel,
        grid=(A, NBLK),
        in_specs=[
            pl.BlockSpec((Q, 128), lambda a, j: (0, 0)),
            pl.BlockSpec((128, D), lambda a, j: (0, 0)),
            pl.BlockSpec((D,), lambda a, j: (0,)),
            pl.BlockSpec((1, D, BC), lambda a, j: (a, 0, j)),
        ],
        out_specs=[
            pl.BlockSpec((1, Q, BC), lambda a, j: (a, 0, j)),
            pl.BlockSpec((1, 1, Q, GPAD), lambda a, j: (a, j, 0, 0)),
        ],
        out_shape=[
            jax.ShapeDtypeStruct((A, Q, CPAD), jnp.float32),
            jax.ShapeDtypeStruct((A, NBLK, Q, GPAD), jnp.float32),
        ],
    )(observations, W_emb, b_emb, keys_t)


def _vt_kernel(vt_ref, out_ref):
    v = jnp.swapaxes(vt_ref[0], 0, 1)
    out_ref[0] = jnp.concatenate(
        [v, jnp.zeros((BC, 128 - D), jnp.float32)], axis=1)


def _transpose_values(dnd_values):
    vals_t = jnp.swapaxes(dnd_values, 1, 2)
    return pl.pallas_call(
        _vt_kernel,
        grid=(A, NBLK),
        in_specs=[pl.BlockSpec((1, D, BC), lambda a, j: (a, 0, j))],
        out_specs=pl.BlockSpec((1, BC, 128), lambda a, j: (a, j, 0)),
        out_shape=jax.ShapeDtypeStruct((A, CPAD, 128), jnp.float32),
    )(vals_t)


def _scal(x):
    return x if getattr(x, "ndim", 0) == 0 else jnp.max(x)


def _ffs(b):
    return _scal(plsc.all_reduce_ffs(b))


def _sc_body(d2_hbm, gm_hbm, vals_hbm, out_hbm,
             cand_v, gm_v, cm1_v, cm_v, vidx_v, vbuf, obuf, wbuf_v,
             selc_smem, sem, sem2):
    lane = lax.iota(jnp.int32, 16)
    lane0 = lane == 0

    def set1f(ref, pos, val):
        plsc.store_scatter(ref, [jnp.full((16,), pos, jnp.int32)],
                           jnp.full((16,), val, jnp.float32), mask=lane0)

    def set1i(ref, pos, val):
        plsc.store_scatter(ref, [jnp.full((16,), pos, jnp.int32)],
                           jnp.full((16,), val, jnp.int32), mask=lane0)

    wid = lax.axis_index("s") * 2 + lax.axis_index("c")

    def row_body(r, _):
        row = wid * RPW + r
        a = row >> 6
        q = row & (Q - 1)

        for j in range(NBLK):
            pltpu.sync_copy(gm_hbm.at[a, j, q],
                            gm_v.at[pl.ds(j * GPAD, GPAD)])

        for v in range(NG // 16):
            set1f(cm1_v, v, jnp.min(gm_v[pl.ds(v * 16, 16)]))

        def ext1(i, _c):
            v0 = cm1_v[pl.ds(0, 16)]
            v1 = cm1_v[pl.ds(16, 16)]
            v2 = cm1_v[pl.ds(32, 16)]
            v3 = cm1_v[pl.ds(48, 16)]
            m = jnp.min(jnp.minimum(jnp.minimum(v0, v1),
                                    jnp.minimum(v2, v3)))
            f0 = _ffs(v0 == m)
            f1 = _ffs(v1 == m)
            f2 = _ffs(v2 == m)
            f3 = _ffs(v3 == m)
            s3 = jnp.where(f0 < 16, f0,
                           jnp.where(f1 < 16, 16 + f1,
                                     jnp.where(f2 < 16, 32 + f2, 48 + f3)))
            gv = gm_v[pl.ds(s3 * 16, 16)]
            l1 = _ffs(gv == m)
            gidx = s3 * 16 + l1
            jb = gidx >> 7
            u = gidx & (GPAD - 1)
            cb = jb * BC + u * 128
            selc_smem[i] = cb
            pltpu.async_copy(d2_hbm.at[a, q, pl.ds(cb, 128)],
                             cand_v.at[i], sem)
            set1f(cm_v, i, m)
            set1f(gm_v, gidx, BIG)
            set1f(cm1_v, s3, jnp.min(gm_v[pl.ds(s3 * 16, 16)]))
            return 0

        lax.fori_loop(0, SEL, ext1, 0)

        for v in range(SEL // 16):
            vidx_v[pl.ds(v * 16, 16)] = a * CPAD + lane + v * 16
            wbuf_v[pl.ds(v * 16, 16)] = jnp.zeros((16,), jnp.float32)

        pltpu.make_async_copy(d2_hbm.at[0, pl.ds(0, SEL), pl.ds(0, 128)],
                              cand_v, sem).wait()

        def ext2(i, _c):
            c0 = cm_v[pl.ds(0, 16)]
            c1 = cm_v[pl.ds(16, 16)]
            c2 = cm_v[pl.ds(32, 16)]
            c3 = cm_v[pl.ds(48, 16)]
            m = jnp.min(jnp.minimum(jnp.minimum(c0, c1),
                                    jnp.minimum(c2, c3)))
            f0 = _ffs(c0 == m)
            f1 = _ffs(c1 == m)
            f2 = _ffs(c2 == m)
            f3 = _ffs(c3 == m)
            s = jnp.where(f0 < 16, f0,
                          jnp.where(f1 < 16, 16 + f1,
                                    jnp.where(f2 < 16, 32 + f2, 48 + f3)))
            cb = selc_smem[s]
            srow = jnp.full((16,), s, jnp.int32)
            col = jnp.int32(-1)
            for p in range(8):
                cv = plsc.load_gather(cand_v, [srow, lane + p * 16])
                fp = _ffs(cv == m)
                col = jnp.where((col < 0) & (fp < 16), p * 16 + fp, col)
            c_local = cb + col
            v_glob = a * CPAD + jnp.minimum(c_local, C - 1)
            set1i(vidx_v, i, v_glob)
            wv = 1.0 / (jnp.full((16,), jnp.maximum(m, 0.0) + DELTA,
                                 jnp.float32))
            plsc.store_scatter(wbuf_v, [jnp.full((16,), i, jnp.int32)],
                               wv, mask=lane0)
            plsc.store_scatter(cand_v, [srow, jnp.full((16,), col, jnp.int32)],
                               jnp.full((16,), BIG, jnp.float32), mask=lane0)
            nm = plsc.load_gather(cand_v, [srow, lane])
            for p in range(1, 8):
                nm = jnp.minimum(nm,
                                 plsc.load_gather(cand_v, [srow, lane + p * 16]))
            set1f(cm_v, s, jnp.min(nm))
            return 0

        lax.fori_loop(0, K, ext2, 0)

        pltpu.async_copy(vals_hbm.at[vidx_v], vbuf, sem).wait()

        def wsloop(i, acc):
            irow = jnp.full((16,), i, jnp.int32)
            wv = plsc.load_gather(wbuf_v, [irow])
            return tuple(
                acc[dv] + wv * plsc.load_gather(vbuf,
                                                [irow, lane + dv * 16])
                for dv in range(4))

        z = jnp.zeros((16,), jnp.float32)
        acc = lax.fori_loop(0, SEL, wsloop, (z, z, z, z))
        ws4 = (wbuf_v[pl.ds(0, 16)] + wbuf_v[pl.ds(16, 16)]
               + wbuf_v[pl.ds(32, 16)] + wbuf_v[pl.ds(48, 16)])
        invv = 1.0 / jnp.full((16,), jnp.sum(ws4), jnp.float32)
        for dv in range(4):
            obuf[pl.ds(dv * 16, 16)] = acc[dv] * invv
        pltpu.sync_copy(obuf, out_hbm.at[pl.ds((q * A + a) * D, D)])
        return 0

    lax.fori_loop(0, RPW, row_body, 0)


def _sc_topk(d2, gm, vals):
    f = functools.partial(
        pl.kernel,
        mesh=plsc.VectorSubcoreMesh(core_axis_name="c", subcore_axis_name="s"),
        compiler_params=pltpu.CompilerParams(needs_layout_passes=False,
                                             use_tc_tiling_on_sc=True),
        out_type=jax.ShapeDtypeStruct((Q * A * D,), jnp.float32),
        scratch_types=[
            pltpu.VMEM((SEL, 128), jnp.float32),
            pltpu.VMEM((NG,), jnp.float32),
            pltpu.VMEM((NG // 16,), jnp.float32),
            pltpu.VMEM((SEL,), jnp.float32),
            pltpu.VMEM((SEL,), jnp.int32),
            pltpu.VMEM((SEL, 128), jnp.float32),
            pltpu.VMEM((D,), jnp.float32),
            pltpu.VMEM((SEL,), jnp.float32),
            pltpu.SMEM((SEL,), jnp.int32),
            pltpu.SemaphoreType.DMA,
            pltpu.SemaphoreType.DMA,
        ],
    )(_sc_body)
    return f(d2, gm, vals)


def kernel(observations, W_emb, b_emb, dnd_keys, dnd_values):
    d2, gm = _distances(observations, W_emb, b_emb, dnd_keys)
    vals_tr = _transpose_values(dnd_values)
    out = _sc_topk(d2, gm, vals_tr.reshape(A * CPAD, 128))
    return out.reshape(Q, A, D)
